# Initial kernel scaffold; baseline (speedup 1.0000x reference)
import jax
import jax.numpy as jnp
from jax.experimental import pallas as pl


def kernel(x, edge_index, W1, b1, W2, b2, W3, b3):
    raise NotImplementedError("write your pallas kernel here")


# speedup vs baseline: 9.4274x; 9.4274x over previous
import functools

import jax
import jax.numpy as jnp
from jax import lax
from jax.experimental import pallas as pl
from jax.experimental.pallas import tpu as pltpu
from jax.experimental.pallas import tpu_sc as plsc

N = 10000
E = 320000
D_IN = 128
HID = 128
CLS = 64

NC = 2
NS = 16
NW = NC * NS

EPT = E // NW
K = 80
NCH = EPT // K
RPT = 624
REM = N - RPT * NS


@functools.partial(jax.jit, static_argnames=("d",))
def _propagate(y, src, dst, zero, d):
    mesh = plsc.VectorSubcoreMesh(
        core_axis_name="c", subcore_axis_name="s", num_cores=NC, num_subcores=NS
    )

    @functools.partial(
        pl.kernel,
        out_type=jax.ShapeDtypeStruct((NC, N, d), jnp.float32),
        mesh=mesh,
        scratch_types=[
            pltpu.VMEM((K,), jnp.int32),
            pltpu.VMEM((K,), jnp.int32),
            pltpu.VMEM((K, d), jnp.float32),
            pltpu.VMEM_SHARED((N, d), jnp.float32),
            pltpu.SemaphoreType.DMA,
        ],
        compiler_params=pltpu.CompilerParams(use_tc_tiling_on_sc=False),
    )
    def prop(y_hbm, src_hbm, dst_hbm, zero_hbm, out_hbm, src_v, dst_v, rows_v, acc_sh, sem):
        cid = lax.axis_index("c")
        sid = lax.axis_index("s")
        wid = cid * NS + sid
        r0 = sid * RPT
        pltpu.sync_copy(zero_hbm.at[pl.ds(r0, RPT)], acc_sh.at[pl.ds(r0, RPT)])

        @pl.when(sid == NS - 1)
        def _():
            pltpu.sync_copy(
                zero_hbm.at[pl.ds(RPT * NS, REM)], acc_sh.at[pl.ds(RPT * NS, REM)]
            )

        plsc.subcore_barrier()

        base0 = wid * EPT

        def body(i, carry):
            base = pl.multiple_of(base0 + i * K, 8)
            pltpu.sync_copy(src_hbm.at[pl.ds(base, K)], src_v)
            pltpu.sync_copy(dst_hbm.at[pl.ds(base, K)], dst_v)
            pltpu.async_copy(y_hbm.at[src_v], rows_v, sem).wait()
            pltpu.sync_copy(rows_v, acc_sh.at[dst_v], add=True)
            return carry

        lax.fori_loop(0, NCH, body, 0)
        plsc.subcore_barrier()
        pltpu.sync_copy(acc_sh.at[pl.ds(r0, RPT)], out_hbm.at[cid, pl.ds(r0, RPT)])

        @pl.when(sid == NS - 1)
        def _():
            pltpu.sync_copy(
                acc_sh.at[pl.ds(RPT * NS, REM)], out_hbm.at[cid, pl.ds(RPT * NS, REM)]
            )

    return prop(y, src, dst, zero)


R = 1000


def _mm_first_body(x_ref, w_ref, dg_ref, y_ref, dinv_ref):
    dg = dg_ref[0] + dg_ref[1] + 1.0
    dinv = lax.rsqrt(dg)
    xw = jnp.dot(x_ref[...], w_ref[...], preferred_element_type=jnp.float32)
    y_ref[...] = xw * dinv
    dinv_ref[...] = dinv


def _mm_first(x, W, dg):
    grid = (N // R,)
    return pl.pallas_call(
        _mm_first_body,
        grid=grid,
        in_specs=[
            pl.BlockSpec((R, x.shape[1]), lambda i: (i, 0)),
            pl.BlockSpec(W.shape, lambda i: (0, 0)),
            pl.BlockSpec((2, R, 1), lambda i: (0, i, 0)),
        ],
        out_specs=[
            pl.BlockSpec((R, W.shape[1]), lambda i: (i, 0)),
            pl.BlockSpec((R, 1), lambda i: (i, 0)),
        ],
        out_shape=[
            jax.ShapeDtypeStruct((N, W.shape[1]), jnp.float32),
            jax.ShapeDtypeStruct((N, 1), jnp.float32),
        ],
    )(x, W, dg)


def _mm_mid_body(a_ref, y_ref, dinv_ref, w_ref, b_ref, o_ref):
    dinv = dinv_ref[...]
    t = (a_ref[0] + a_ref[1] + y_ref[...]) * dinv + b_ref[...]
    h = jnp.maximum(t, 0.0)
    o_ref[...] = jnp.dot(h, w_ref[...], preferred_element_type=jnp.float32) * dinv


def _mm_mid(a, y, dinv, W, b):
    d = y.shape[1]
    grid = (N // R,)
    return pl.pallas_call(
        _mm_mid_body,
        grid=grid,
        in_specs=[
            pl.BlockSpec((2, R, d), lambda i: (0, i, 0)),
            pl.BlockSpec((R, d), lambda i: (i, 0)),
            pl.BlockSpec((R, 1), lambda i: (i, 0)),
            pl.BlockSpec(W.shape, lambda i: (0, 0)),
            pl.BlockSpec((1, d), lambda i: (0, 0)),
        ],
        out_specs=pl.BlockSpec((R, W.shape[1]), lambda i: (i, 0)),
        out_shape=jax.ShapeDtypeStruct((N, W.shape[1]), jnp.float32),
    )(a, y, dinv, W, b.reshape(1, -1))


def _mm_last_body(a_ref, y_ref, dinv_ref, b_ref, o_ref):
    t = (a_ref[0] + a_ref[1] + y_ref[...]) * dinv_ref[...] + b_ref[...]
    m = jnp.max(t, axis=1, keepdims=True)
    t = t - m
    lse = jnp.log(jnp.sum(jnp.exp(t), axis=1, keepdims=True))
    o_ref[...] = t - lse


def _mm_last(a, y, dinv, b):
    d = y.shape[1]
    grid = (N // R,)
    return pl.pallas_call(
        _mm_last_body,
        grid=grid,
        in_specs=[
            pl.BlockSpec((2, R, d), lambda i: (0, i, 0)),
            pl.BlockSpec((R, d), lambda i: (i, 0)),
            pl.BlockSpec((R, 1), lambda i: (i, 0)),
            pl.BlockSpec((1, d), lambda i: (0, 0)),
        ],
        out_specs=pl.BlockSpec((R, d), lambda i: (i, 0)),
        out_shape=jax.ShapeDtypeStruct((N, d), jnp.float32),
    )(a, y, dinv, b.reshape(1, -1))


def kernel(x, edge_index, W1, b1, W2, b2, W3, b3):
    src = edge_index[0]
    dst = edge_index[1]

    ones128 = jnp.ones((N, HID), jnp.float32)
    zeros128 = jnp.zeros((N, HID), jnp.float32)
    zeros64 = jnp.zeros((N, CLS), jnp.float32)

    degp = _propagate(ones128, src, dst, zeros128, HID)
    dg = degp[:, :, :1]

    y1, dinv = _mm_first(x, W1, dg)
    a1 = _propagate(y1, src, dst, zeros128, HID)
    y2 = _mm_mid(a1, y1, dinv, W2, b1)
    a2 = _propagate(y2, src, dst, zeros128, HID)
    y3 = _mm_mid(a2, y2, dinv, W3, b2)
    a3 = _propagate(y3, src, dst, zeros64, CLS)
    return _mm_last(a3, y3, dinv, b3)


# speedup vs baseline: 30.8725x; 3.2748x over previous
import functools

import jax
import jax.numpy as jnp
from jax import lax
from jax.experimental import pallas as pl
from jax.experimental.pallas import tpu as pltpu
from jax.experimental.pallas import tpu_sc as plsc

N = 10000
E = 320000
D_IN = 128
HID = 128
CLS = 64

NC = 2
NS = 16
NW = NC * NS

EPT = E // NW
K = 40
NCH = EPT // K
NB = 5
GA = 4
RPT = 624
REM = N - RPT * NS

_MESH = dict(core_axis_name="c", subcore_axis_name="s", num_cores=NC,
             num_subcores=NS)
_SC_PARAMS = pltpu.CompilerParams(use_tc_tiling_on_sc=False)


@functools.partial(jax.jit, static_argnames=("d",))
def _propagate(y, srcb, dstb, zero, d):
    @functools.partial(
        pl.kernel,
        out_type=jax.ShapeDtypeStruct((NC, N, d), jnp.float32),
        mesh=plsc.VectorSubcoreMesh(**_MESH),
        scratch_types=[
            pltpu.VMEM((NCH, K), jnp.int32),
            pltpu.VMEM((NCH, K), jnp.int32),
            pltpu.VMEM((NB, K, d), jnp.float32),
            pltpu.VMEM_SHARED((N, d), jnp.float32),
        ] + [pltpu.SemaphoreType.DMA] * NB,
        compiler_params=_SC_PARAMS,
    )
    def prop(y_hbm, srcb_hbm, dstb_hbm, zero_hbm, out_hbm,
             srcb_v, dstb_v, rows_v, acc_sh, *sems):
        cid = lax.axis_index("c")
        sid = lax.axis_index("s")
        wid = cid * NS + sid
        r0 = sid * RPT
        pltpu.sync_copy(zero_hbm.at[pl.ds(r0, RPT)], acc_sh.at[pl.ds(r0, RPT)])

        @pl.when(sid == NS - 1)
        def _():
            pltpu.sync_copy(
                zero_hbm.at[pl.ds(RPT * NS, REM)], acc_sh.at[pl.ds(RPT * NS, REM)]
            )

        pltpu.sync_copy(srcb_hbm.at[wid], srcb_v)
        pltpu.sync_copy(dstb_hbm.at[wid], dstb_v)
        plsc.subcore_barrier()

        for b in range(GA):
            pltpu.async_copy(y_hbm.at[srcb_v.at[b]], rows_v.at[b], sems[b])

        @pl.loop(0, NCH // NB)
        def _(g):
            i0 = g * NB
            for b in range(NB):
                i = i0 + b
                bg = (b + GA) % NB

                @pl.when(i + GA < NCH)
                def _():
                    pltpu.async_copy(
                        y_hbm.at[srcb_v.at[i + GA]], rows_v.at[bg], sems[bg]
                    )

                pltpu.make_async_copy(
                    y_hbm.at[srcb_v.at[b]], rows_v.at[b], sems[b]
                ).wait()
                pltpu.sync_copy(rows_v.at[b], acc_sh.at[dstb_v.at[i]], add=True)

        plsc.subcore_barrier()
        pltpu.sync_copy(acc_sh.at[pl.ds(r0, RPT)], out_hbm.at[cid, pl.ds(r0, RPT)])

        @pl.when(sid == NS - 1)
        def _():
            pltpu.sync_copy(
                acc_sh.at[pl.ds(RPT * NS, REM)], out_hbm.at[cid, pl.ds(RPT * NS, REM)]
            )

    return prop(y, srcb, dstb, zero)


@jax.jit
def _degree(dstb, zero, ones):
    @functools.partial(
        pl.kernel,
        out_type=jax.ShapeDtypeStruct((NC, N, 16), jnp.float32),
        mesh=plsc.VectorSubcoreMesh(**_MESH),
        scratch_types=[
            pltpu.VMEM((NCH, K), jnp.int32),
            pltpu.VMEM((K, 16), jnp.float32),
            pltpu.VMEM_SHARED((N, 16), jnp.float32),
        ],
        compiler_params=_SC_PARAMS,
    )
    def degk(dstb_hbm, zero_hbm, ones_hbm, out_hbm, dstb_v, ones_v, acc_sh):
        cid = lax.axis_index("c")
        sid = lax.axis_index("s")
        wid = cid * NS + sid
        r0 = sid * RPT
        pltpu.sync_copy(zero_hbm.at[pl.ds(r0, RPT)], acc_sh.at[pl.ds(r0, RPT)])

        @pl.when(sid == NS - 1)
        def _():
            pltpu.sync_copy(
                zero_hbm.at[pl.ds(RPT * NS, REM)], acc_sh.at[pl.ds(RPT * NS, REM)]
            )

        pltpu.sync_copy(ones_hbm, ones_v)
        pltpu.sync_copy(dstb_hbm.at[wid], dstb_v)
        plsc.subcore_barrier()

        @pl.loop(0, NCH)
        def _(i):
            pltpu.sync_copy(ones_v, acc_sh.at[dstb_v.at[i]], add=True)

        plsc.subcore_barrier()
        pltpu.sync_copy(acc_sh.at[pl.ds(r0, RPT)], out_hbm.at[cid, pl.ds(r0, RPT)])

        @pl.when(sid == NS - 1)
        def _():
            pltpu.sync_copy(
                acc_sh.at[pl.ds(RPT * NS, REM)], out_hbm.at[cid, pl.ds(RPT * NS, REM)]
            )

    return degk(dstb, zero, ones)


R = 1000


def _mm_first_body(x_ref, w_ref, dg_ref, y_ref, dinv_ref):
    dg = dg_ref[0] + dg_ref[1] + 1.0
    dinv = lax.rsqrt(dg)
    xw = jnp.dot(x_ref[...], w_ref[...], preferred_element_type=jnp.float32)
    y_ref[...] = xw * dinv
    dinv_ref[...] = dinv


def _mm_first(x, W, dg):
    grid = (N // R,)
    return pl.pallas_call(
        _mm_first_body,
        grid=grid,
        in_specs=[
            pl.BlockSpec((R, x.shape[1]), lambda i: (i, 0)),
            pl.BlockSpec(W.shape, lambda i: (0, 0)),
            pl.BlockSpec((2, R, 1), lambda i: (0, i, 0)),
        ],
        out_specs=[
            pl.BlockSpec((R, W.shape[1]), lambda i: (i, 0)),
            pl.BlockSpec((R, 1), lambda i: (i, 0)),
        ],
        out_shape=[
            jax.ShapeDtypeStruct((N, W.shape[1]), jnp.float32),
            jax.ShapeDtypeStruct((N, 1), jnp.float32),
        ],
    )(x, W, dg)


def _mm_mid_body(a_ref, y_ref, dinv_ref, w_ref, b_ref, o_ref):
    dinv = dinv_ref[...]
    t = (a_ref[0] + a_ref[1] + y_ref[...]) * dinv + b_ref[...]
    h = jnp.maximum(t, 0.0)
    o_ref[...] = jnp.dot(h, w_ref[...], preferred_element_type=jnp.float32) * dinv


def _mm_mid(a, y, dinv, W, b):
    d = y.shape[1]
    grid = (N // R,)
    return pl.pallas_call(
        _mm_mid_body,
        grid=grid,
        in_specs=[
            pl.BlockSpec((2, R, d), lambda i: (0, i, 0)),
            pl.BlockSpec((R, d), lambda i: (i, 0)),
            pl.BlockSpec((R, 1), lambda i: (i, 0)),
            pl.BlockSpec(W.shape, lambda i: (0, 0)),
            pl.BlockSpec((1, d), lambda i: (0, 0)),
        ],
        out_specs=pl.BlockSpec((R, W.shape[1]), lambda i: (i, 0)),
        out_shape=jax.ShapeDtypeStruct((N, W.shape[1]), jnp.float32),
    )(a, y, dinv, W, b.reshape(1, -1))


def _mm_last_body(a_ref, y_ref, dinv_ref, b_ref, o_ref):
    t = (a_ref[0] + a_ref[1] + y_ref[...]) * dinv_ref[...] + b_ref[...]
    m = jnp.max(t, axis=1, keepdims=True)
    t = t - m
    lse = jnp.log(jnp.sum(jnp.exp(t), axis=1, keepdims=True))
    o_ref[...] = t - lse


def _mm_last(a, y, dinv, b):
    d = y.shape[1]
    grid = (N // R,)
    return pl.pallas_call(
        _mm_last_body,
        grid=grid,
        in_specs=[
            pl.BlockSpec((2, R, d), lambda i: (0, i, 0)),
            pl.BlockSpec((R, d), lambda i: (i, 0)),
            pl.BlockSpec((R, 1), lambda i: (i, 0)),
            pl.BlockSpec((1, d), lambda i: (0, 0)),
        ],
        out_specs=pl.BlockSpec((R, d), lambda i: (i, 0)),
        out_shape=jax.ShapeDtypeStruct((N, d), jnp.float32),
    )(a, y, dinv, b.reshape(1, -1))


def kernel(x, edge_index, W1, b1, W2, b2, W3, b3):
    srcb = edge_index[0].reshape(NW, NCH, K)
    dstb = edge_index[1].reshape(NW, NCH, K)

    zeros16 = jnp.zeros((N, 16), jnp.float32)
    ones16 = jnp.ones((K, 16), jnp.float32)
    zeros128 = jnp.zeros((N, HID), jnp.float32)
    zeros64 = jnp.zeros((N, CLS), jnp.float32)

    degp = _degree(dstb, zeros16, ones16)
    dg = degp[:, :, :1]

    y1, dinv = _mm_first(x, W1, dg)
    a1 = _propagate(y1, srcb, dstb, zeros128, HID)
    y2 = _mm_mid(a1, y1, dinv, W2, b1)
    a2 = _propagate(y2, srcb, dstb, zeros128, HID)
    y3 = _mm_mid(a2, y2, dinv, W3, b2)
    a3 = _propagate(y3, srcb, dstb, zeros64, CLS)
    return _mm_last(a3, y3, dinv, b3)


# speedup vs baseline: 30.9236x; 1.0017x over previous
import functools

import jax
import jax.numpy as jnp
from jax import lax
from jax.experimental import pallas as pl
from jax.experimental.pallas import tpu as pltpu
from jax.experimental.pallas import tpu_sc as plsc

N = 10000
E = 320000
D_IN = 128
HID = 128
CLS = 64

NC = 2
NS = 16
NW = NC * NS

EPT = E // NW
K = 40
NCH = EPT // K
NB = 5
GA = 3
KD = 80
NCHD = EPT // KD
RPT = 624
REM = N - RPT * NS

_MESH = dict(core_axis_name="c", subcore_axis_name="s", num_cores=NC,
             num_subcores=NS)
_SC_PARAMS = pltpu.CompilerParams(use_tc_tiling_on_sc=False)


def _acc_init(zero_hbm, acc_sh, sid):
    r0 = sid * RPT
    pltpu.sync_copy(zero_hbm.at[pl.ds(r0, RPT)], acc_sh.at[pl.ds(r0, RPT)])

    @pl.when(sid == NS - 1)
    def _():
        pltpu.sync_copy(
            zero_hbm.at[pl.ds(RPT * NS, REM)], acc_sh.at[pl.ds(RPT * NS, REM)]
        )


def _acc_readout(acc_sh, out_hbm, cid, sid):
    r0 = sid * RPT
    pltpu.sync_copy(acc_sh.at[pl.ds(r0, RPT)], out_hbm.at[cid, pl.ds(r0, RPT)])

    @pl.when(sid == NS - 1)
    def _():
        pltpu.sync_copy(
            acc_sh.at[pl.ds(RPT * NS, REM)], out_hbm.at[cid, pl.ds(RPT * NS, REM)]
        )


@functools.partial(jax.jit, static_argnames=("d",))
def _propagate(y, srcb, dstb, zero, d):
    @functools.partial(
        pl.kernel,
        out_type=jax.ShapeDtypeStruct((NC, N, d), jnp.float32),
        mesh=plsc.VectorSubcoreMesh(**_MESH),
        scratch_types=[
            pltpu.VMEM((NCH, K), jnp.int32),
            pltpu.VMEM((NCH, K), jnp.int32),
            pltpu.VMEM((NB, K, d), jnp.float32),
            pltpu.VMEM_SHARED((N, d), jnp.float32),
        ] + [pltpu.SemaphoreType.DMA] * (2 * NB),
        compiler_params=_SC_PARAMS,
    )
    def prop(y_hbm, srcb_hbm, dstb_hbm, zero_hbm, out_hbm,
             srcb_v, dstb_v, rows_v, acc_sh, *sems):
        gsems = sems[:NB]
        ssems = sems[NB:]
        cid = lax.axis_index("c")
        sid = lax.axis_index("s")
        wid = cid * NS + sid
        _acc_init(zero_hbm, acc_sh, sid)
        pltpu.sync_copy(srcb_hbm.at[wid], srcb_v)
        pltpu.sync_copy(dstb_hbm.at[wid], dstb_v)
        plsc.subcore_barrier()

        for b in range(GA):
            pltpu.async_copy(y_hbm.at[srcb_v.at[b]], rows_v.at[b], gsems[b])

        @pl.loop(0, NCH // NB)
        def _(g):
            i0 = g * NB
            for b in range(NB):
                i = i0 + b
                bg = (b + GA) % NB

                @pl.when(i + GA < NCH)
                def _():
                    @pl.when(i + GA - NB >= 0)
                    def _():
                        pltpu.make_async_copy(
                            rows_v.at[bg], acc_sh.at[dstb_v.at[0]], ssems[bg]
                        ).wait()

                    pltpu.async_copy(
                        y_hbm.at[srcb_v.at[i + GA]], rows_v.at[bg], gsems[bg]
                    )

                pltpu.make_async_copy(
                    y_hbm.at[srcb_v.at[0]], rows_v.at[b], gsems[b]
                ).wait()
                pltpu.async_copy(
                    rows_v.at[b], acc_sh.at[dstb_v.at[i]], ssems[b], add=True
                )

        for b in range(NB):
            pltpu.make_async_copy(
                rows_v.at[b], acc_sh.at[dstb_v.at[0]], ssems[b]
            ).wait()
        plsc.subcore_barrier()
        _acc_readout(acc_sh, out_hbm, cid, sid)

    return prop(y, srcb, dstb, zero)


@jax.jit
def _degree(dstb, zero, ones):
    @functools.partial(
        pl.kernel,
        out_type=jax.ShapeDtypeStruct((NC, N, 16), jnp.float32),
        mesh=plsc.VectorSubcoreMesh(**_MESH),
        scratch_types=[
            pltpu.VMEM((NCHD, KD), jnp.int32),
            pltpu.VMEM((KD, 16), jnp.float32),
            pltpu.VMEM_SHARED((N, 16), jnp.float32),
        ] + [pltpu.SemaphoreType.DMA] * NB,
        compiler_params=_SC_PARAMS,
    )
    def degk(dstb_hbm, zero_hbm, ones_hbm, out_hbm, dstb_v, ones_v, acc_sh,
             *ssems):
        cid = lax.axis_index("c")
        sid = lax.axis_index("s")
        wid = cid * NS + sid
        _acc_init(zero_hbm, acc_sh, sid)
        pltpu.sync_copy(ones_hbm, ones_v)
        pltpu.sync_copy(dstb_hbm.at[wid], dstb_v)
        plsc.subcore_barrier()

        @pl.loop(0, NCHD // NB)
        def _(g):
            i0 = g * NB
            for b in range(NB):
                i = i0 + b

                @pl.when(i >= NB)
                def _():
                    pltpu.make_async_copy(
                        ones_v, acc_sh.at[dstb_v.at[0]], ssems[b]
                    ).wait()

                pltpu.async_copy(
                    ones_v, acc_sh.at[dstb_v.at[i]], ssems[b], add=True
                )

        for b in range(NB):
            pltpu.make_async_copy(ones_v, acc_sh.at[dstb_v.at[0]], ssems[b]).wait()
        plsc.subcore_barrier()
        _acc_readout(acc_sh, out_hbm, cid, sid)

    return degk(dstb, zero, ones)


R = 1000


def _mm_xw_body(x_ref, w_ref, o_ref):
    o_ref[...] = jnp.dot(x_ref[...], w_ref[...], preferred_element_type=jnp.float32)


def _mm_xw(x, W):
    grid = (N // R,)
    return pl.pallas_call(
        _mm_xw_body,
        grid=grid,
        in_specs=[
            pl.BlockSpec((R, x.shape[1]), lambda i: (i, 0)),
            pl.BlockSpec(W.shape, lambda i: (0, 0)),
        ],
        out_specs=pl.BlockSpec((R, W.shape[1]), lambda i: (i, 0)),
        out_shape=jax.ShapeDtypeStruct((N, W.shape[1]), jnp.float32),
    )(x, W)


def _mm_scale_body(xw_ref, dg_ref, y_ref, dinv_ref):
    dinv = lax.rsqrt(dg_ref[0] + dg_ref[1] + 1.0)
    y_ref[...] = xw_ref[...] * dinv
    dinv_ref[...] = dinv


def _mm_scale(xw, dg):
    d = xw.shape[1]
    grid = (N // R,)
    return pl.pallas_call(
        _mm_scale_body,
        grid=grid,
        in_specs=[
            pl.BlockSpec((R, d), lambda i: (i, 0)),
            pl.BlockSpec((2, R, 1), lambda i: (0, i, 0)),
        ],
        out_specs=[
            pl.BlockSpec((R, d), lambda i: (i, 0)),
            pl.BlockSpec((R, 1), lambda i: (i, 0)),
        ],
        out_shape=[
            jax.ShapeDtypeStruct((N, d), jnp.float32),
            jax.ShapeDtypeStruct((N, 1), jnp.float32),
        ],
    )(xw, dg)


def _mm_mid_body(a_ref, y_ref, dinv_ref, w_ref, b_ref, o_ref):
    dinv = dinv_ref[...]
    t = (a_ref[0] + a_ref[1] + y_ref[...]) * dinv + b_ref[...]
    h = jnp.maximum(t, 0.0)
    o_ref[...] = jnp.dot(h, w_ref[...], preferred_element_type=jnp.float32) * dinv


def _mm_mid(a, y, dinv, W, b):
    d = y.shape[1]
    grid = (N // R,)
    return pl.pallas_call(
        _mm_mid_body,
        grid=grid,
        in_specs=[
            pl.BlockSpec((2, R, d), lambda i: (0, i, 0)),
            pl.BlockSpec((R, d), lambda i: (i, 0)),
            pl.BlockSpec((R, 1), lambda i: (i, 0)),
            pl.BlockSpec(W.shape, lambda i: (0, 0)),
            pl.BlockSpec((1, d), lambda i: (0, 0)),
        ],
        out_specs=pl.BlockSpec((R, W.shape[1]), lambda i: (i, 0)),
        out_shape=jax.ShapeDtypeStruct((N, W.shape[1]), jnp.float32),
    )(a, y, dinv, W, b.reshape(1, -1))


def _mm_last_body(a_ref, y_ref, dinv_ref, b_ref, o_ref):
    t = (a_ref[0] + a_ref[1] + y_ref[...]) * dinv_ref[...] + b_ref[...]
    m = jnp.max(t, axis=1, keepdims=True)
    t = t - m
    lse = jnp.log(jnp.sum(jnp.exp(t), axis=1, keepdims=True))
    o_ref[...] = t - lse


def _mm_last(a, y, dinv, b):
    d = y.shape[1]
    grid = (N // R,)
    return pl.pallas_call(
        _mm_last_body,
        grid=grid,
        in_specs=[
            pl.BlockSpec((2, R, d), lambda i: (0, i, 0)),
            pl.BlockSpec((R, d), lambda i: (i, 0)),
            pl.BlockSpec((R, 1), lambda i: (i, 0)),
            pl.BlockSpec((1, d), lambda i: (0, 0)),
        ],
        out_specs=pl.BlockSpec((R, d), lambda i: (i, 0)),
        out_shape=jax.ShapeDtypeStruct((N, d), jnp.float32),
    )(a, y, dinv, b.reshape(1, -1))


def kernel(x, edge_index, W1, b1, W2, b2, W3, b3):
    srcb = edge_index[0].reshape(NW, NCH, K)
    dstb = edge_index[1].reshape(NW, NCH, K)
    dstbd = edge_index[1].reshape(NW, NCHD, KD)

    zeros16 = jnp.zeros((N, 16), jnp.float32)
    ones16 = jnp.ones((KD, 16), jnp.float32)
    zeros128 = jnp.zeros((N, HID), jnp.float32)
    zeros64 = jnp.zeros((N, CLS), jnp.float32)

    degp = _degree(dstbd, zeros16, ones16)
    xw1 = _mm_xw(x, W1)
    dg = degp[:, :, :1]

    y1, dinv = _mm_scale(xw1, dg)
    a1 = _propagate(y1, srcb, dstb, zeros128, HID)
    y2 = _mm_mid(a1, y1, dinv, W2, b1)
    a2 = _propagate(y2, srcb, dstb, zeros128, HID)
    y3 = _mm_mid(a2, y2, dinv, W3, b2)
    a3 = _propagate(y3, srcb, dstb, zeros64, CLS)
    return _mm_last(a3, y3, dinv, b3)


# speedup vs baseline: 32.2686x; 1.0435x over previous
import functools

import jax
import jax.numpy as jnp
from jax import lax
from jax.experimental import pallas as pl
from jax.experimental.pallas import tpu as pltpu
from jax.experimental.pallas import tpu_sc as plsc

N = 10000
E = 320000
D_IN = 128
HID = 128
CLS = 64

NC = 2
NS = 16
NW = NC * NS

EPT = E // NW
K = 40
NCH = EPT // K
NB = 5
GA = 4
KD = 80
NCHD = EPT // KD
RPT = 624
REM = N - RPT * NS

_MESH = dict(core_axis_name="c", subcore_axis_name="s", num_cores=NC,
             num_subcores=NS)
_SC_PARAMS = pltpu.CompilerParams(use_tc_tiling_on_sc=False)


def _acc_init(zero_hbm, acc_sh, sid):
    r0 = sid * RPT
    pltpu.sync_copy(zero_hbm.at[pl.ds(r0, RPT)], acc_sh.at[pl.ds(r0, RPT)])

    @pl.when(sid == NS - 1)
    def _():
        pltpu.sync_copy(
            zero_hbm.at[pl.ds(RPT * NS, REM)], acc_sh.at[pl.ds(RPT * NS, REM)]
        )


def _acc_readout(acc_sh, out_hbm, cid, sid):
    r0 = sid * RPT
    pltpu.sync_copy(acc_sh.at[pl.ds(r0, RPT)], out_hbm.at[cid, pl.ds(r0, RPT)])

    @pl.when(sid == NS - 1)
    def _():
        pltpu.sync_copy(
            acc_sh.at[pl.ds(RPT * NS, REM)], out_hbm.at[cid, pl.ds(RPT * NS, REM)]
        )


@functools.partial(jax.jit, static_argnames=("d",))
def _propagate(y, srcb, dstb, zero, d):
    @functools.partial(
        pl.kernel,
        out_type=jax.ShapeDtypeStruct((NC, N, d), jnp.float32),
        mesh=plsc.VectorSubcoreMesh(**_MESH),
        scratch_types=[
            pltpu.VMEM((NCH, K), jnp.int32),
            pltpu.VMEM((NCH, K), jnp.int32),
            pltpu.VMEM((NB, K, d), jnp.float32),
            pltpu.VMEM_SHARED((N, d), jnp.float32),
        ] + [pltpu.SemaphoreType.DMA] * NB,
        compiler_params=_SC_PARAMS,
    )
    def prop(y_hbm, srcb_hbm, dstb_hbm, zero_hbm, out_hbm,
             srcb_v, dstb_v, rows_v, acc_sh, *gsems):
        cid = lax.axis_index("c")
        sid = lax.axis_index("s")
        wid = cid * NS + sid
        pltpu.sync_copy(srcb_hbm.at[wid], srcb_v)
        pltpu.sync_copy(dstb_hbm.at[wid], dstb_v)
        for b in range(GA):
            pltpu.async_copy(y_hbm.at[srcb_v.at[b]], rows_v.at[b], gsems[b])
        _acc_init(zero_hbm, acc_sh, sid)
        plsc.subcore_barrier()

        @pl.loop(0, NCH // NB)
        def _(g):
            i0 = g * NB
            for b in range(NB):
                i = i0 + b
                bg = (b + GA) % NB

                @pl.when(i + GA < NCH)
                def _():
                    pltpu.async_copy(
                        y_hbm.at[srcb_v.at[i + GA]], rows_v.at[bg], gsems[bg]
                    )

                pltpu.make_async_copy(
                    y_hbm.at[srcb_v.at[0]], rows_v.at[b], gsems[b]
                ).wait()
                pltpu.sync_copy(rows_v.at[b], acc_sh.at[dstb_v.at[i]], add=True)

        plsc.subcore_barrier()
        _acc_readout(acc_sh, out_hbm, cid, sid)

    return prop(y, srcb, dstb, zero)


@jax.jit
def _degree(dstb, zero, ones):
    @functools.partial(
        pl.kernel,
        out_type=jax.ShapeDtypeStruct((NC, N, 16), jnp.float32),
        mesh=plsc.VectorSubcoreMesh(**_MESH),
        scratch_types=[
            pltpu.VMEM((NCHD, KD), jnp.int32),
            pltpu.VMEM((KD, 16), jnp.float32),
            pltpu.VMEM_SHARED((N, 16), jnp.float32),
        ] + [pltpu.SemaphoreType.DMA] * NB,
        compiler_params=_SC_PARAMS,
    )
    def degk(dstb_hbm, zero_hbm, ones_hbm, out_hbm, dstb_v, ones_v, acc_sh,
             *ssems):
        cid = lax.axis_index("c")
        sid = lax.axis_index("s")
        wid = cid * NS + sid
        _acc_init(zero_hbm, acc_sh, sid)
        pltpu.sync_copy(ones_hbm, ones_v)
        pltpu.sync_copy(dstb_hbm.at[wid], dstb_v)
        plsc.subcore_barrier()

        @pl.loop(0, NCHD // NB)
        def _(g):
            i0 = g * NB
            for b in range(NB):
                i = i0 + b

                @pl.when(i >= NB)
                def _():
                    pltpu.make_async_copy(
                        ones_v, acc_sh.at[dstb_v.at[0]], ssems[b]
                    ).wait()

                pltpu.async_copy(
                    ones_v, acc_sh.at[dstb_v.at[i]], ssems[b], add=True
                )

        for b in range(NB):
            pltpu.make_async_copy(ones_v, acc_sh.at[dstb_v.at[0]], ssems[b]).wait()
        plsc.subcore_barrier()
        _acc_readout(acc_sh, out_hbm, cid, sid)

    return degk(dstb, zero, ones)


R = 1000


def _mm_xw_body(x_ref, w_ref, o_ref):
    o_ref[...] = jnp.dot(x_ref[...], w_ref[...], preferred_element_type=jnp.float32)


def _mm_xw(x, W):
    grid = (N // R,)
    return pl.pallas_call(
        _mm_xw_body,
        grid=grid,
        in_specs=[
            pl.BlockSpec((R, x.shape[1]), lambda i: (i, 0)),
            pl.BlockSpec(W.shape, lambda i: (0, 0)),
        ],
        out_specs=pl.BlockSpec((R, W.shape[1]), lambda i: (i, 0)),
        out_shape=jax.ShapeDtypeStruct((N, W.shape[1]), jnp.float32),
    )(x, W)


def _mm_scale_body(xw_ref, dg_ref, y_ref, dinv_ref):
    dinv = lax.rsqrt(dg_ref[0] + dg_ref[1] + 1.0)
    y_ref[...] = xw_ref[...] * dinv
    dinv_ref[...] = dinv


def _mm_scale(xw, dg):
    d = xw.shape[1]
    grid = (N // R,)
    return pl.pallas_call(
        _mm_scale_body,
        grid=grid,
        in_specs=[
            pl.BlockSpec((R, d), lambda i: (i, 0)),
            pl.BlockSpec((2, R, 1), lambda i: (0, i, 0)),
        ],
        out_specs=[
            pl.BlockSpec((R, d), lambda i: (i, 0)),
            pl.BlockSpec((R, 1), lambda i: (i, 0)),
        ],
        out_shape=[
            jax.ShapeDtypeStruct((N, d), jnp.float32),
            jax.ShapeDtypeStruct((N, 1), jnp.float32),
        ],
    )(xw, dg)


def _mm_mid_body(a_ref, y_ref, dinv_ref, w_ref, b_ref, o_ref):
    dinv = dinv_ref[...]
    t = (a_ref[0] + a_ref[1] + y_ref[...]) * dinv + b_ref[...]
    h = jnp.maximum(t, 0.0)
    o_ref[...] = jnp.dot(h, w_ref[...], preferred_element_type=jnp.float32) * dinv


def _mm_mid(a, y, dinv, W, b):
    d = y.shape[1]
    grid = (N // R,)
    return pl.pallas_call(
        _mm_mid_body,
        grid=grid,
        in_specs=[
            pl.BlockSpec((2, R, d), lambda i: (0, i, 0)),
            pl.BlockSpec((R, d), lambda i: (i, 0)),
            pl.BlockSpec((R, 1), lambda i: (i, 0)),
            pl.BlockSpec(W.shape, lambda i: (0, 0)),
            pl.BlockSpec((1, d), lambda i: (0, 0)),
        ],
        out_specs=pl.BlockSpec((R, W.shape[1]), lambda i: (i, 0)),
        out_shape=jax.ShapeDtypeStruct((N, W.shape[1]), jnp.float32),
    )(a, y, dinv, W, b.reshape(1, -1))


def _mm_last_body(a_ref, y_ref, dinv_ref, b_ref, o_ref):
    t = (a_ref[0] + a_ref[1] + y_ref[...]) * dinv_ref[...] + b_ref[...]
    m = jnp.max(t, axis=1, keepdims=True)
    t = t - m
    lse = jnp.log(jnp.sum(jnp.exp(t), axis=1, keepdims=True))
    o_ref[...] = t - lse


def _mm_last(a, y, dinv, b):
    d = y.shape[1]
    grid = (N // R,)
    return pl.pallas_call(
        _mm_last_body,
        grid=grid,
        in_specs=[
            pl.BlockSpec((2, R, d), lambda i: (0, i, 0)),
            pl.BlockSpec((R, d), lambda i: (i, 0)),
            pl.BlockSpec((R, 1), lambda i: (i, 0)),
            pl.BlockSpec((1, d), lambda i: (0, 0)),
        ],
        out_specs=pl.BlockSpec((R, d), lambda i: (i, 0)),
        out_shape=jax.ShapeDtypeStruct((N, d), jnp.float32),
    )(a, y, dinv, b.reshape(1, -1))


def kernel(x, edge_index, W1, b1, W2, b2, W3, b3):
    srcb = edge_index[0].reshape(NW, NCH, K)
    dstb = edge_index[1].reshape(NW, NCH, K)
    dstbd = edge_index[1].reshape(NW, NCHD, KD)

    zeros16 = jnp.zeros((N, 16), jnp.float32)
    ones16 = jnp.ones((KD, 16), jnp.float32)
    zeros128 = jnp.zeros((N, HID), jnp.float32)
    zeros64 = jnp.zeros((N, CLS), jnp.float32)

    degp = _degree(dstbd, zeros16, ones16)
    xw1 = _mm_xw(x, W1)
    dg = degp[:, :, :1]

    y1, dinv = _mm_scale(xw1, dg)
    a1 = _propagate(y1, srcb, dstb, zeros128, HID)
    y2 = _mm_mid(a1, y1, dinv, W2, b1)
    a2 = _propagate(y2, srcb, dstb, zeros128, HID)
    y3 = _mm_mid(a2, y2, dinv, W3, b2)
    a3 = _propagate(y3, srcb, dstb, zeros64, CLS)
    return _mm_last(a3, y3, dinv, b3)


# speedup vs baseline: 33.2005x; 1.0289x over previous
import functools

import jax
import jax.numpy as jnp
from jax import lax
from jax.experimental import pallas as pl
from jax.experimental.pallas import tpu as pltpu
from jax.experimental.pallas import tpu_sc as plsc

N = 10000
E = 320000
D_IN = 128
HID = 128
CLS = 64

NC = 2
NS = 16
NW = NC * NS

EPT = E // NW
K = 40
NCH = EPT // K
NB = 5
GA = 4
KD = 80
NCHD = EPT // KD
RPT = 624
REM = N - RPT * NS

_MESH = dict(core_axis_name="c", subcore_axis_name="s", num_cores=NC,
             num_subcores=NS)
_SC_PARAMS = pltpu.CompilerParams(use_tc_tiling_on_sc=False)


def _acc_init(zero_hbm, acc_sh, sid):
    r0 = sid * RPT
    pltpu.sync_copy(zero_hbm.at[pl.ds(r0, RPT)], acc_sh.at[pl.ds(r0, RPT)])

    @pl.when(sid == NS - 1)
    def _():
        pltpu.sync_copy(
            zero_hbm.at[pl.ds(RPT * NS, REM)], acc_sh.at[pl.ds(RPT * NS, REM)]
        )


def _acc_readout(acc_sh, out_hbm, cid, sid):
    r0 = sid * RPT
    pltpu.sync_copy(acc_sh.at[pl.ds(r0, RPT)], out_hbm.at[cid, pl.ds(r0, RPT)])

    @pl.when(sid == NS - 1)
    def _():
        pltpu.sync_copy(
            acc_sh.at[pl.ds(RPT * NS, REM)], out_hbm.at[cid, pl.ds(RPT * NS, REM)]
        )


@functools.partial(jax.jit, static_argnames=("d",))
def _propagate(y, srcb, dstb, zero, d):
    k = srcb.shape[2]
    nch = srcb.shape[1]

    @functools.partial(
        pl.kernel,
        out_type=jax.ShapeDtypeStruct((NC, N, d), jnp.float32),
        mesh=plsc.VectorSubcoreMesh(**_MESH),
        scratch_types=[
            pltpu.VMEM((nch, k), jnp.int32),
            pltpu.VMEM((nch, k), jnp.int32),
            pltpu.VMEM((NB, k, d), jnp.float32),
            pltpu.VMEM_SHARED((N, d), jnp.float32),
        ] + [pltpu.SemaphoreType.DMA] * NB,
        compiler_params=_SC_PARAMS,
    )
    def prop(y_hbm, srcb_hbm, dstb_hbm, zero_hbm, out_hbm,
             srcb_v, dstb_v, rows_v, acc_sh, *gsems):
        cid = lax.axis_index("c")
        sid = lax.axis_index("s")
        wid = cid * NS + sid
        pltpu.sync_copy(srcb_hbm.at[wid], srcb_v)
        pltpu.sync_copy(dstb_hbm.at[wid], dstb_v)
        for b in range(GA):
            pltpu.async_copy(y_hbm.at[srcb_v.at[b]], rows_v.at[b], gsems[b])
        _acc_init(zero_hbm, acc_sh, sid)
        plsc.subcore_barrier()

        @pl.loop(0, nch // NB)
        def _(g):
            i0 = g * NB
            for b in range(NB):
                i = i0 + b
                bg = (b + GA) % NB

                @pl.when(i + GA < nch)
                def _():
                    pltpu.async_copy(
                        y_hbm.at[srcb_v.at[i + GA]], rows_v.at[bg], gsems[bg]
                    )

                pltpu.make_async_copy(
                    y_hbm.at[srcb_v.at[0]], rows_v.at[b], gsems[b]
                ).wait()
                pltpu.sync_copy(rows_v.at[b], acc_sh.at[dstb_v.at[i]], add=True)

        plsc.subcore_barrier()
        _acc_readout(acc_sh, out_hbm, cid, sid)

    return prop(y, srcb, dstb, zero)


@jax.jit
def _degree(dstb, zero, ones):
    @functools.partial(
        pl.kernel,
        out_type=jax.ShapeDtypeStruct((NC, N, 16), jnp.float32),
        mesh=plsc.VectorSubcoreMesh(**_MESH),
        scratch_types=[
            pltpu.VMEM((NCHD, KD), jnp.int32),
            pltpu.VMEM((KD, 16), jnp.float32),
            pltpu.VMEM_SHARED((N, 16), jnp.float32),
        ] + [pltpu.SemaphoreType.DMA] * NB,
        compiler_params=_SC_PARAMS,
    )
    def degk(dstb_hbm, zero_hbm, ones_hbm, out_hbm, dstb_v, ones_v, acc_sh,
             *ssems):
        cid = lax.axis_index("c")
        sid = lax.axis_index("s")
        wid = cid * NS + sid
        _acc_init(zero_hbm, acc_sh, sid)
        pltpu.sync_copy(ones_hbm, ones_v)
        pltpu.sync_copy(dstb_hbm.at[wid], dstb_v)
        plsc.subcore_barrier()

        @pl.loop(0, NCHD // NB)
        def _(g):
            i0 = g * NB
            for b in range(NB):
                i = i0 + b

                @pl.when(i >= NB)
                def _():
                    pltpu.make_async_copy(
                        ones_v, acc_sh.at[dstb_v.at[0]], ssems[b]
                    ).wait()

                pltpu.async_copy(
                    ones_v, acc_sh.at[dstb_v.at[i]], ssems[b], add=True
                )

        for b in range(NB):
            pltpu.make_async_copy(ones_v, acc_sh.at[dstb_v.at[0]], ssems[b]).wait()
        plsc.subcore_barrier()
        _acc_readout(acc_sh, out_hbm, cid, sid)

    return degk(dstb, zero, ones)


R = 1000


def _mm_first_body(x_ref, w_ref, dg_ref, y_ref, dinv_ref):
    dinv = lax.rsqrt(dg_ref[0, :, 0:1] + dg_ref[1, :, 0:1] + 1.0)
    xw = jnp.dot(x_ref[...], w_ref[...], preferred_element_type=jnp.float32)
    y_ref[...] = xw * dinv
    dinv_ref[...] = dinv


def _mm_first(x, W, degp):
    grid = (N // R,)
    return pl.pallas_call(
        _mm_first_body,
        grid=grid,
        in_specs=[
            pl.BlockSpec((R, x.shape[1]), lambda i: (i, 0)),
            pl.BlockSpec(W.shape, lambda i: (0, 0)),
            pl.BlockSpec((2, R, 16), lambda i: (0, i, 0)),
        ],
        out_specs=[
            pl.BlockSpec((R, W.shape[1]), lambda i: (i, 0)),
            pl.BlockSpec((R, 1), lambda i: (i, 0)),
        ],
        out_shape=[
            jax.ShapeDtypeStruct((N, W.shape[1]), jnp.float32),
            jax.ShapeDtypeStruct((N, 1), jnp.float32),
        ],
    )(x, W, degp)


def _mm_mid_body(a_ref, y_ref, dinv_ref, w_ref, b_ref, o_ref):
    dinv = dinv_ref[...]
    t = (a_ref[0] + a_ref[1] + y_ref[...]) * dinv + b_ref[...]
    h = jnp.maximum(t, 0.0)
    o_ref[...] = jnp.dot(h, w_ref[...], preferred_element_type=jnp.float32) * dinv


def _mm_mid(a, y, dinv, W, b):
    d = y.shape[1]
    grid = (N // R,)
    return pl.pallas_call(
        _mm_mid_body,
        grid=grid,
        in_specs=[
            pl.BlockSpec((2, R, d), lambda i: (0, i, 0)),
            pl.BlockSpec((R, d), lambda i: (i, 0)),
            pl.BlockSpec((R, 1), lambda i: (i, 0)),
            pl.BlockSpec(W.shape, lambda i: (0, 0)),
            pl.BlockSpec((1, d), lambda i: (0, 0)),
        ],
        out_specs=pl.BlockSpec((R, W.shape[1]), lambda i: (i, 0)),
        out_shape=jax.ShapeDtypeStruct((N, W.shape[1]), jnp.float32),
    )(a, y, dinv, W, b.reshape(1, -1))


def _mm_last_body(a_ref, y_ref, dinv_ref, b_ref, o_ref):
    t = (a_ref[0] + a_ref[1] + y_ref[...]) * dinv_ref[...] + b_ref[...]
    m = jnp.max(t, axis=1, keepdims=True)
    t = t - m
    lse = jnp.log(jnp.sum(jnp.exp(t), axis=1, keepdims=True))
    o_ref[...] = t - lse


def _mm_last(a, y, dinv, b):
    d = y.shape[1]
    grid = (N // R,)
    return pl.pallas_call(
        _mm_last_body,
        grid=grid,
        in_specs=[
            pl.BlockSpec((2, R, d), lambda i: (0, i, 0)),
            pl.BlockSpec((R, d), lambda i: (i, 0)),
            pl.BlockSpec((R, 1), lambda i: (i, 0)),
            pl.BlockSpec((1, d), lambda i: (0, 0)),
        ],
        out_specs=pl.BlockSpec((R, d), lambda i: (i, 0)),
        out_shape=jax.ShapeDtypeStruct((N, d), jnp.float32),
    )(a, y, dinv, b.reshape(1, -1))


def kernel(x, edge_index, W1, b1, W2, b2, W3, b3):
    srcb = edge_index[0].reshape(NW, NCH, K)
    dstb = edge_index[1].reshape(NW, NCH, K)
    srcb8 = edge_index[0].reshape(NW, NCHD, KD)
    dstb8 = edge_index[1].reshape(NW, NCHD, KD)

    zeros16 = jnp.zeros((N, 16), jnp.float32)
    ones16 = jnp.ones((KD, 16), jnp.float32)
    zeros128 = jnp.zeros((N, HID), jnp.float32)
    zeros64 = jnp.zeros((N, CLS), jnp.float32)

    degp = _degree(dstb8, zeros16, ones16)
    y1, dinv = _mm_first(x, W1, degp)
    a1 = _propagate(y1, srcb, dstb, zeros128, HID)
    y2 = _mm_mid(a1, y1, dinv, W2, b1)
    a2 = _propagate(y2, srcb, dstb, zeros128, HID)
    y3 = _mm_mid(a2, y2, dinv, W3, b2)
    a3 = _propagate(y3, srcb8, dstb8, zeros64, CLS)
    return _mm_last(a3, y3, dinv, b3)
